# Initial kernel scaffold; baseline (speedup 1.0000x reference)
import jax
import jax.numpy as jnp
from jax.experimental import pallas as pl


def kernel(hidden_state, in_proj_w, in_proj_b, out_proj_w, out_proj_b, fn1_w, fn1_b, ln1_g, ln1_b, fn2_w, fn2_b, ln2_g, ln2_b, ctx_w, ctx_b, ln3_g, ln3_b, rh1_w, rh1_b, ln4_g, ln4_b, rh2_w, rh2_b, exp_w, exp_b, dh1_w, dh1_b, ln5_g, ln5_b, dh2_w, dh2_b):
    raise NotImplementedError("write your pallas kernel here")


# speedup vs baseline: 2.1013x; 2.1013x over previous
import functools

import jax
import jax.numpy as jnp
from jax import lax
from jax.experimental import pallas as pl
from jax.experimental.pallas import tpu as pltpu

_NH = 8
_EPS = 1e-5


def _attn_mean_body(x_ref, w_ref, b_ref, wo_ref, bo_ref, o_ref, *, hd):
    x = x_ref[0]
    w = w_ref[...]
    b = b_ref[...]
    qkv = jnp.dot(x, w, preferred_element_type=jnp.float32) + b
    s_len = x.shape[0]
    h = x.shape[1]
    sqrt_d = jnp.sqrt(jnp.float32(hd))
    parts = []
    for n in range(_NH):
        q = qkv[:, n * hd:(n + 1) * hd]
        k = qkv[:, h + n * hd:h + (n + 1) * hd]
        v = qkv[:, 2 * h + n * hd:2 * h + (n + 1) * hd]
        s = lax.dot_general(q, k, (((1,), (1,)), ((), ())),
                            preferred_element_type=jnp.float32) / sqrt_d
        m = jnp.max(s, axis=1, keepdims=True)
        e = jnp.exp(s - m)
        r = jnp.sum(e, axis=1, keepdims=True)
        parts.append(jnp.dot(e, v, preferred_element_type=jnp.float32) / r)
    ao = jnp.concatenate(parts, axis=1)
    attn_out = jnp.dot(ao, wo_ref[...],
                       preferred_element_type=jnp.float32) + bo_ref[...]
    o_ref[0] = jnp.mean(attn_out, axis=0, keepdims=True)


def _ln(x, g, b):
    m = jnp.mean(x, axis=-1, keepdims=True)
    v = jnp.mean((x - m) ** 2, axis=-1, keepdims=True)
    return (x - m) * lax.rsqrt(v + _EPS) * g + b


def _gelu(x):
    return x * 0.5 * (1.0 + lax.erf(x * (2.0 ** -0.5)))


def _head_body(seq_ref, cls_ref,
               f1w_ref, f1b_ref, g1_ref, b1_ref,
               f2w_ref, f2b_ref, g2_ref, b2_ref,
               cw_ref, cb_ref, g3_ref, b3_ref,
               r1w_ref, r1b_ref, g4_ref, b4_ref,
               r2w_ref, r2b_ref,
               d1w_ref, d1b_ref, g5_ref, b5_ref,
               d2w_ref, d2b_ref,
               e0w_ref, e0b_ref, e1w_ref, e1b_ref,
               fin_ref, gp_ref, el0_ref, el1_ref, dom_ref):
    dot = lambda a, b: jnp.dot(a, b, preferred_element_type=jnp.float32)
    seq = seq_ref[...]
    cls = cls_ref[...]
    f = _gelu(_ln(dot(seq, f1w_ref[...]) + f1b_ref[...], g1_ref[...], b1_ref[...]))
    f = _gelu(_ln(dot(f, f2w_ref[...]) + f2b_ref[...], g2_ref[...], b2_ref[...]))
    ctx = _gelu(_ln(dot(f, cw_ref[...]) + cb_ref[...], g3_ref[...], b3_ref[...]))
    r = _gelu(_ln(dot(ctx, r1w_ref[...]) + r1b_ref[...], g4_ref[...], b4_ref[...]))
    gl = dot(r, r2w_ref[...]) + r2b_ref[...]
    gm = jnp.max(gl, axis=1, keepdims=True)
    ge = jnp.exp(gl - gm)
    gp = ge / jnp.sum(ge, axis=1, keepdims=True)
    gp_ref[...] = gp

    d = _gelu(_ln(dot(cls, d1w_ref[...]) + d1b_ref[...], g5_ref[...], b5_ref[...]))
    dom_ref[...] = dot(d, d2w_ref[...]) + d2b_ref[...]

    al0 = dot(cls, e0w_ref[...]) + e0b_ref[...]
    al1 = dot(cls, e1w_ref[...]) + e1b_ref[...]

    n_e = gp.shape[1]
    idx = lax.broadcasted_iota(jnp.int32, gp.shape, 1)
    m1 = jnp.max(gp, axis=1, keepdims=True)
    i1 = jnp.min(jnp.where(gp == m1, idx, n_e), axis=1, keepdims=True)
    mask1 = idx == i1
    gp2 = jnp.where(mask1, -1.0, gp)
    m2 = jnp.max(gp2, axis=1, keepdims=True)
    i2 = jnp.min(jnp.where(gp2 == m2, idx, n_e), axis=1, keepdims=True)
    mask2 = idx == i2
    denom = m1 + m2
    w1 = m1 / denom
    w2 = m2 / denom
    s1_0 = jnp.sum(jnp.where(mask1, al0, 0.0), axis=1, keepdims=True)
    s1_1 = jnp.sum(jnp.where(mask1, al1, 0.0), axis=1, keepdims=True)
    s2_0 = jnp.sum(jnp.where(mask2, al0, 0.0), axis=1, keepdims=True)
    s2_1 = jnp.sum(jnp.where(mask2, al1, 0.0), axis=1, keepdims=True)
    fin_ref[...] = jnp.concatenate(
        [w1 * s1_0 + w2 * s2_0, w1 * s1_1 + w2 * s2_1], axis=1)
    mboth = mask1 | mask2
    el0_ref[...] = jnp.where(mboth, al0, 0.0)
    el1_ref[...] = jnp.where(mboth, al1, 0.0)


def kernel(hidden_state, in_proj_w, in_proj_b, out_proj_w, out_proj_b,
           fn1_w, fn1_b, ln1_g, ln1_b, fn2_w, fn2_b, ln2_g, ln2_b,
           ctx_w, ctx_b, ln3_g, ln3_b, rh1_w, rh1_b, ln4_g, ln4_b,
           rh2_w, rh2_b, exp_w, exp_b, dh1_w, dh1_b, ln5_g, ln5_b,
           dh2_w, dh2_b):
    b, s, h = hidden_state.shape
    hd = h // _NH
    e_num, l_num, _ = exp_w.shape

    seq_repr = pl.pallas_call(
        functools.partial(_attn_mean_body, hd=hd),
        grid=(b,),
        in_specs=[
            pl.BlockSpec((1, s, h), lambda i: (i, 0, 0)),
            pl.BlockSpec((h, 3 * h), lambda i: (0, 0)),
            pl.BlockSpec((1, 3 * h), lambda i: (0, 0)),
            pl.BlockSpec((h, h), lambda i: (0, 0)),
            pl.BlockSpec((1, h), lambda i: (0, 0)),
        ],
        out_specs=pl.BlockSpec((1, 1, h), lambda i: (i, 0, 0)),
        out_shape=jax.ShapeDtypeStruct((b, 1, h), jnp.float32),
        compiler_params=pltpu.CompilerParams(
            dimension_semantics=("arbitrary",)),
    )(hidden_state, in_proj_w.T, in_proj_b.reshape(1, 3 * h),
      out_proj_w.T, out_proj_b.reshape(1, h))
    seq_repr = seq_repr.reshape(b, h)

    cls = hidden_state[:, 0, :]
    row = lambda t: t.reshape(1, -1)
    outs = pl.pallas_call(
        _head_body,
        out_shape=[
            jax.ShapeDtypeStruct((b, l_num), jnp.float32),
            jax.ShapeDtypeStruct((b, e_num), jnp.float32),
            jax.ShapeDtypeStruct((b, e_num), jnp.float32),
            jax.ShapeDtypeStruct((b, e_num), jnp.float32),
            jax.ShapeDtypeStruct((b, dh2_w.shape[0]), jnp.float32),
        ],
    )(seq_repr, cls,
      fn1_w.T, row(fn1_b), row(ln1_g), row(ln1_b),
      fn2_w.T, row(fn2_b), row(ln2_g), row(ln2_b),
      ctx_w.T, row(ctx_b), row(ln3_g), row(ln3_b),
      rh1_w.T, row(rh1_b), row(ln4_g), row(ln4_b),
      rh2_w.T, row(rh2_b),
      dh1_w.T, row(dh1_b), row(ln5_g), row(ln5_b),
      dh2_w.T, row(dh2_b),
      exp_w[:, 0, :].T, row(exp_b[:, 0]),
      exp_w[:, 1, :].T, row(exp_b[:, 1]))

    final_logits, gating_probs, el0, el1, domain_logits = outs
    expert_logits = jnp.stack([el0, el1], axis=-1)
    return (final_logits, gating_probs, expert_logits, domain_logits)


# speedup vs baseline: 2.2032x; 1.0485x over previous
import functools

import jax
import jax.numpy as jnp
from jax import lax
from jax.experimental import pallas as pl
from jax.experimental.pallas import tpu as pltpu

_NH = 8
_EPS = 1e-5


def _attn_mean_body(x_ref, w_ref, b_ref, wo_ref, bo_ref, o_ref, *, hd, bb):
    s_len = x_ref.shape[1]
    h = x_ref.shape[2]
    x = x_ref[...].reshape(bb * s_len, h)
    qkv_all = jnp.dot(x, w_ref[...],
                      preferred_element_type=jnp.float32) + b_ref[...]
    sqrt_d = jnp.sqrt(jnp.float32(hd))
    for i in range(bb):
        qkv = qkv_all[i * s_len:(i + 1) * s_len]
        parts = []
        for n in range(_NH):
            q = qkv[:, n * hd:(n + 1) * hd]
            k = qkv[:, h + n * hd:h + (n + 1) * hd]
            v = qkv[:, 2 * h + n * hd:2 * h + (n + 1) * hd]
            s = lax.dot_general(q, k, (((1,), (1,)), ((), ())),
                                preferred_element_type=jnp.float32) / sqrt_d
            m = jnp.max(s, axis=1, keepdims=True)
            e = jnp.exp(s - m)
            r = jnp.sum(e, axis=1, keepdims=True)
            parts.append(jnp.dot(e, v, preferred_element_type=jnp.float32) / r)
        ao = jnp.concatenate(parts, axis=1)
        attn_out = jnp.dot(ao, wo_ref[...],
                           preferred_element_type=jnp.float32) + bo_ref[...]
        o_ref[i] = jnp.mean(attn_out, axis=0, keepdims=True)


def _ln(x, g, b):
    m = jnp.mean(x, axis=-1, keepdims=True)
    v = jnp.mean((x - m) ** 2, axis=-1, keepdims=True)
    return (x - m) * lax.rsqrt(v + _EPS) * g + b


def _gelu(x):
    return x * 0.5 * (1.0 + lax.erf(x * (2.0 ** -0.5)))


def _head_body(seq_ref, cls_ref,
               f1w_ref, f1b_ref, g1_ref, b1_ref,
               f2w_ref, f2b_ref, g2_ref, b2_ref,
               cw_ref, cb_ref, g3_ref, b3_ref,
               r1w_ref, r1b_ref, g4_ref, b4_ref,
               r2w_ref, r2b_ref,
               d1w_ref, d1b_ref, g5_ref, b5_ref,
               d2w_ref, d2b_ref,
               e0w_ref, e0b_ref, e1w_ref, e1b_ref,
               fin_ref, gp_ref, el0_ref, el1_ref, dom_ref):
    dot = lambda a, b: jnp.dot(a, b, preferred_element_type=jnp.float32)
    seq = seq_ref[...]
    cls = cls_ref[...]
    f = _gelu(_ln(dot(seq, f1w_ref[...]) + f1b_ref[...], g1_ref[...], b1_ref[...]))
    f = _gelu(_ln(dot(f, f2w_ref[...]) + f2b_ref[...], g2_ref[...], b2_ref[...]))
    ctx = _gelu(_ln(dot(f, cw_ref[...]) + cb_ref[...], g3_ref[...], b3_ref[...]))
    r = _gelu(_ln(dot(ctx, r1w_ref[...]) + r1b_ref[...], g4_ref[...], b4_ref[...]))
    gl = dot(r, r2w_ref[...]) + r2b_ref[...]
    gm = jnp.max(gl, axis=1, keepdims=True)
    ge = jnp.exp(gl - gm)
    gp = ge / jnp.sum(ge, axis=1, keepdims=True)
    gp_ref[...] = gp

    d = _gelu(_ln(dot(cls, d1w_ref[...]) + d1b_ref[...], g5_ref[...], b5_ref[...]))
    dom_ref[...] = dot(d, d2w_ref[...]) + d2b_ref[...]

    al0 = dot(cls, e0w_ref[...]) + e0b_ref[...]
    al1 = dot(cls, e1w_ref[...]) + e1b_ref[...]

    n_e = gp.shape[1]
    idx = lax.broadcasted_iota(jnp.int32, gp.shape, 1)
    m1 = jnp.max(gp, axis=1, keepdims=True)
    i1 = jnp.min(jnp.where(gp == m1, idx, n_e), axis=1, keepdims=True)
    mask1 = idx == i1
    gp2 = jnp.where(mask1, -1.0, gp)
    m2 = jnp.max(gp2, axis=1, keepdims=True)
    i2 = jnp.min(jnp.where(gp2 == m2, idx, n_e), axis=1, keepdims=True)
    mask2 = idx == i2
    denom = m1 + m2
    w1 = m1 / denom
    w2 = m2 / denom
    s1_0 = jnp.sum(jnp.where(mask1, al0, 0.0), axis=1, keepdims=True)
    s1_1 = jnp.sum(jnp.where(mask1, al1, 0.0), axis=1, keepdims=True)
    s2_0 = jnp.sum(jnp.where(mask2, al0, 0.0), axis=1, keepdims=True)
    s2_1 = jnp.sum(jnp.where(mask2, al1, 0.0), axis=1, keepdims=True)
    fin_ref[...] = jnp.concatenate(
        [w1 * s1_0 + w2 * s2_0, w1 * s1_1 + w2 * s2_1], axis=1)
    mboth = mask1 | mask2
    el0_ref[...] = jnp.where(mboth, al0, 0.0)
    el1_ref[...] = jnp.where(mboth, al1, 0.0)


def kernel(hidden_state, in_proj_w, in_proj_b, out_proj_w, out_proj_b,
           fn1_w, fn1_b, ln1_g, ln1_b, fn2_w, fn2_b, ln2_g, ln2_b,
           ctx_w, ctx_b, ln3_g, ln3_b, rh1_w, rh1_b, ln4_g, ln4_b,
           rh2_w, rh2_b, exp_w, exp_b, dh1_w, dh1_b, ln5_g, ln5_b,
           dh2_w, dh2_b):
    b, s, h = hidden_state.shape
    hd = h // _NH
    e_num, l_num, _ = exp_w.shape

    bb = 2
    seq_repr = pl.pallas_call(
        functools.partial(_attn_mean_body, hd=hd, bb=bb),
        grid=(b // bb,),
        in_specs=[
            pl.BlockSpec((bb, s, h), lambda i: (i, 0, 0)),
            pl.BlockSpec((h, 3 * h), lambda i: (0, 0)),
            pl.BlockSpec((1, 3 * h), lambda i: (0, 0)),
            pl.BlockSpec((h, h), lambda i: (0, 0)),
            pl.BlockSpec((1, h), lambda i: (0, 0)),
        ],
        out_specs=pl.BlockSpec((bb, 1, h), lambda i: (i, 0, 0)),
        out_shape=jax.ShapeDtypeStruct((b, 1, h), jnp.float32),
        compiler_params=pltpu.CompilerParams(
            dimension_semantics=("arbitrary",)),
    )(hidden_state, in_proj_w.T, in_proj_b.reshape(1, 3 * h),
      out_proj_w.T, out_proj_b.reshape(1, h))
    seq_repr = seq_repr.reshape(b, h)

    cls = hidden_state[:, 0, :]
    row = lambda t: t.reshape(1, -1)
    outs = pl.pallas_call(
        _head_body,
        out_shape=[
            jax.ShapeDtypeStruct((b, l_num), jnp.float32),
            jax.ShapeDtypeStruct((b, e_num), jnp.float32),
            jax.ShapeDtypeStruct((b, e_num), jnp.float32),
            jax.ShapeDtypeStruct((b, e_num), jnp.float32),
            jax.ShapeDtypeStruct((b, dh2_w.shape[0]), jnp.float32),
        ],
    )(seq_repr, cls,
      fn1_w.T, row(fn1_b), row(ln1_g), row(ln1_b),
      fn2_w.T, row(fn2_b), row(ln2_g), row(ln2_b),
      ctx_w.T, row(ctx_b), row(ln3_g), row(ln3_b),
      rh1_w.T, row(rh1_b), row(ln4_g), row(ln4_b),
      rh2_w.T, row(rh2_b),
      dh1_w.T, row(dh1_b), row(ln5_g), row(ln5_b),
      dh2_w.T, row(dh2_b),
      exp_w[:, 0, :].T, row(exp_b[:, 0]),
      exp_w[:, 1, :].T, row(exp_b[:, 1]))

    final_logits, gating_probs, el0, el1, domain_logits = outs
    expert_logits = jnp.stack([el0, el1], axis=-1)
    return (final_logits, gating_probs, expert_logits, domain_logits)
